# baseline (speedup 1.0000x reference)
import jax
import jax.numpy as jnp
from jax import lax
from jax.experimental import pallas as pl
from jax.experimental.pallas import tpu as pltpu
from jax.experimental.pallas import tpu_sc as plsc

H = 256
HP = 512
NUM_GRAPHS = 256
N_NODES = 100000

ROW_BLOCK = 800
N_BLOCKS = N_NODES // ROW_BLOCK
N_PAD = (N_BLOCKS + 1) * ROW_BLOCK
SLAB = 64
NCH = HP // 16
GPW = NUM_GRAPHS // 32


def _mlp_body(x_ref, wu_ref, bu_ref, wg_ref, bg_ref, o_ref):
    i = pl.program_id(0)

    @pl.when(i < N_BLOCKS)
    def _():
        x = x_ref[...].astype(jnp.bfloat16)
        wu = wu_ref[...].astype(jnp.bfloat16)
        wg = wg_ref[...].astype(jnp.bfloat16)
        up = jnp.dot(x, wu, preferred_element_type=jnp.float32) + bu_ref[...]
        gl = jnp.dot(x, wg, preferred_element_type=jnp.float32) + bg_ref[...]
        o_ref[...] = up * (1.0 / (1.0 + jnp.exp(-gl)))

    @pl.when(i >= N_BLOCKS)
    def _():
        o_ref[...] = jnp.zeros_like(o_ref)


def _mlp(x, W_up, b_up, W_gate, b_gate):
    return pl.pallas_call(
        _mlp_body,
        grid=(N_BLOCKS + 1,),
        in_specs=[
            pl.BlockSpec((ROW_BLOCK, H), lambda i: (jnp.minimum(i, N_BLOCKS - 1), 0)),
            pl.BlockSpec((H, HP), lambda i: (0, 0)),
            pl.BlockSpec((1, HP), lambda i: (0, 0)),
            pl.BlockSpec((H, HP), lambda i: (0, 0)),
            pl.BlockSpec((1, HP), lambda i: (0, 0)),
        ],
        out_specs=pl.BlockSpec((ROW_BLOCK, HP), lambda i: (i, 0)),
        out_shape=jax.ShapeDtypeStruct((N_PAD, HP), jnp.float32),
    )(x, W_up, b_up.reshape(1, HP), W_gate, b_gate.reshape(1, HP))


def _accum_rows(buf, lo, hi, carries):
    def row_body(r, cs):
        return tuple(v + buf[r, pl.ds(cc * 16, 16)] for cc, v in enumerate(cs))

    return lax.fori_loop(lo, hi, row_body, carries)


def _sc_body(gated_hbm, starts_hbm, out_hbm, sv, buf_a, buf_b, acc, sem_a, sem_b):
    c = lax.axis_index("c")
    s = lax.axis_index("s")
    w = s * 2 + c

    pltpu.sync_copy(starts_hbm, sv)
    bounds = sv[pl.ds(GPW * w, 16)]

    for j in range(GPW):
        s_j = bounds[j].astype(jnp.int32)
        e_j = bounds[j + 1].astype(jnp.int32)
        a_j = (s_j // 8) * 8
        nslab = (e_j - a_j + SLAB - 1) // SLAB
        npair = (nslab + 1) // 2

        @pl.when(nslab > 0)
        def _(s_j=s_j, e_j=e_j, a_j=a_j, nslab=nslab, npair=npair):
            pltpu.async_copy(gated_hbm.at[pl.ds(a_j, SLAB)], buf_a, sem_a)

            def clip(t):
                base = a_j + t * SLAB
                return (jnp.clip(s_j - base, 0, SLAB),
                        jnp.clip(e_j - base, 0, SLAB))

            def pair_body(u, carries):
                t0 = 2 * u
                t1 = t0 + 1
                t2 = t0 + 2
                pltpu.make_async_copy(gated_hbm.at[pl.ds(a_j, SLAB)],
                                      buf_a, sem_a).wait()

                @pl.when(t1 < nslab)
                def _():
                    pltpu.async_copy(
                        gated_hbm.at[pl.ds(a_j + t1 * SLAB, SLAB)], buf_b, sem_b)

                lo0, hi0 = clip(t0)
                carries = _accum_rows(buf_a, lo0, hi0, carries)

                @pl.when(t2 < nslab)
                def _():
                    pltpu.async_copy(
                        gated_hbm.at[pl.ds(a_j + t2 * SLAB, SLAB)], buf_a, sem_a)

                @pl.when(t1 < nslab)
                def _():
                    pltpu.make_async_copy(gated_hbm.at[pl.ds(a_j, SLAB)],
                                          buf_b, sem_b).wait()

                lo1, hi1 = clip(t1)
                return _accum_rows(buf_b, lo1, hi1, carries)

            carries = lax.fori_loop(
                0, npair, pair_body,
                tuple(jnp.zeros((16,), jnp.float32) for _ in range(NCH)))
            for cc in range(NCH):
                acc[j, pl.ds(cc * 16, 16)] = carries[cc]

        @pl.when(nslab <= 0)
        def _():
            for cc in range(NCH):
                acc[j, pl.ds(cc * 16, 16)] = jnp.zeros((16,), jnp.float32)

    pltpu.sync_copy(acc, out_hbm.at[pl.ds(GPW * w, GPW)])


def _sc_segment_sum(gated, starts):
    mesh = plsc.VectorSubcoreMesh(core_axis_name="c", subcore_axis_name="s",
                                  num_cores=2, num_subcores=16)
    k = pl.kernel(
        _sc_body,
        out_type=jax.ShapeDtypeStruct((NUM_GRAPHS, HP), jnp.float32),
        mesh=mesh,
        scratch_types=[
            pltpu.VMEM((NUM_GRAPHS + 8,), jnp.float32),
            pltpu.VMEM((SLAB, HP), jnp.float32),
            pltpu.VMEM((SLAB, HP), jnp.float32),
            pltpu.VMEM((GPW, HP), jnp.float32),
            pltpu.SemaphoreType.DMA,
            pltpu.SemaphoreType.DMA,
        ],
    )
    return k(gated, starts)


def _final_body(p_ref, w_ref, b_ref, o_ref):
    o_ref[...] = jnp.dot(p_ref[...], w_ref[...],
                         preferred_element_type=jnp.float32) + b_ref[...]


def _final(sums, W_func, b_func):
    return pl.pallas_call(
        _final_body,
        out_shape=jax.ShapeDtypeStruct((NUM_GRAPHS, HP), jnp.float32),
    )(sums, W_func, b_func.reshape(1, HP))


def kernel(node_features, node_to_graph_id, W_up, b_up, W_gate, b_gate, W_func, b_func):
    ids32 = node_to_graph_id.astype(jnp.int32)
    starts = jnp.searchsorted(ids32, jnp.arange(NUM_GRAPHS + 1, dtype=jnp.int32),
                              side="left").astype(jnp.int32)
    starts = jnp.concatenate([starts, jnp.full((7,), N_NODES, jnp.int32)])
    starts = starts.astype(jnp.float32)
    gated = _mlp(node_features, W_up, b_up, W_gate, b_gate)
    sums = _sc_segment_sum(gated, starts)
    return _final(sums, W_func, b_func)


# speedup vs baseline: 1.0025x; 1.0025x over previous
import jax
import jax.numpy as jnp
from jax import lax
from jax.experimental import pallas as pl
from jax.experimental.pallas import tpu as pltpu
from jax.experimental.pallas import tpu_sc as plsc

H = 256
HP = 512
NUM_GRAPHS = 256
N_NODES = 100000
N_PAIRS = N_NODES // 2

ROW_BLOCK = 800
PAIR_BLOCK = ROW_BLOCK // 2
N_BLOCKS = N_NODES // ROW_BLOCK
PS_PAD = (N_BLOCKS + 1) * PAIR_BLOCK
SLAB = 64
NCH = HP // 16
GPW = NUM_GRAPHS // 32
NE = 2 * NUM_GRAPHS


def _gated(x, wu, bu, wg, bg):
    up = jnp.dot(x, wu, preferred_element_type=jnp.float32) + bu
    gl = jnp.dot(x, wg, preferred_element_type=jnp.float32) + bg
    return up * (1.0 / (1.0 + jnp.exp(-gl)))


def _mlp_body(x_ref, wu_ref, bu_ref, wg_ref, bg_ref, o_ref):
    i = pl.program_id(0)

    @pl.when(i < N_BLOCKS)
    def _():
        res = _gated(x_ref[...], wu_ref[...], bu_ref[...],
                     wg_ref[...], bg_ref[...])
        merged = res.reshape(PAIR_BLOCK, 2 * HP)
        o_ref[...] = merged[:, :HP] + merged[:, HP:]

    @pl.when(i >= N_BLOCKS)
    def _():
        o_ref[...] = jnp.zeros_like(o_ref)


def _mlp_pairsum(x, W_up, b_up, W_gate, b_gate):
    return pl.pallas_call(
        _mlp_body,
        grid=(N_BLOCKS + 1,),
        in_specs=[
            pl.BlockSpec((ROW_BLOCK, H), lambda i: (jnp.minimum(i, N_BLOCKS - 1), 0)),
            pl.BlockSpec((H, HP), lambda i: (0, 0)),
            pl.BlockSpec((1, HP), lambda i: (0, 0)),
            pl.BlockSpec((H, HP), lambda i: (0, 0)),
            pl.BlockSpec((1, HP), lambda i: (0, 0)),
        ],
        out_specs=pl.BlockSpec((PAIR_BLOCK, HP), lambda i: (i, 0)),
        out_shape=jax.ShapeDtypeStruct((PS_PAD, HP), jnp.float32),
    )(x, W_up, b_up.reshape(1, HP), W_gate, b_gate.reshape(1, HP))


def _edge_body(x_ref, wu_ref, bu_ref, wg_ref, bg_ref, m_ref, o_ref):
    res = _gated(x_ref[...], wu_ref[...], bu_ref[...], wg_ref[...], bg_ref[...])
    o_ref[...] = res * m_ref[...]


def _edge_mlp(x_e, W_up, b_up, W_gate, b_gate, mask):
    return pl.pallas_call(
        _edge_body,
        out_shape=jax.ShapeDtypeStruct((NE, HP), jnp.float32),
    )(x_e, W_up, b_up.reshape(1, HP), W_gate, b_gate.reshape(1, HP), mask)


def _accum_rows(buf, lo, hi, carries):
    def row_body(r, cs):
        return tuple(v + buf[r, pl.ds(cc * 16, 16)] for cc, v in enumerate(cs))

    return lax.fori_loop(lo, hi, row_body, carries)


def _sc_body(ps_hbm, edges_hbm, starts_hbm, out_hbm,
             sv, buf_a, buf_b, ebuf, acc, sem_a, sem_b):
    c = lax.axis_index("c")
    s = lax.axis_index("s")
    w = s * 2 + c

    pltpu.sync_copy(starts_hbm, sv)
    pltpu.sync_copy(edges_hbm.at[pl.ds(16 * w, 16)], ebuf)
    bounds = sv[pl.ds(GPW * w, 16)]

    for j in range(GPW):
        s_j = bounds[j].astype(jnp.int32)
        e_j = bounds[j + 1].astype(jnp.int32)
        p_s = (s_j + 1) // 2
        p_e = e_j // 2
        a_j = (p_s // 8) * 8
        nslab = (p_e - a_j + SLAB - 1) // SLAB
        npair = (nslab + 1) // 2

        def clip(t, p_s=p_s, p_e=p_e, a_j=a_j):
            base = a_j + t * SLAB
            return (jnp.clip(p_s - base, 0, SLAB),
                    jnp.clip(p_e - base, 0, SLAB))

        carries = tuple(
            ebuf[2 * j, pl.ds(cc * 16, 16)] + ebuf[2 * j + 1, pl.ds(cc * 16, 16)]
            for cc in range(NCH))

        @pl.when(nslab > 0)
        def _(a_j=a_j, nslab=nslab, npair=npair, clip=clip, carries=carries, j=j):
            pltpu.async_copy(ps_hbm.at[pl.ds(a_j, SLAB)], buf_a, sem_a)

            def pair_body(u, carries):
                t0 = 2 * u
                t1 = t0 + 1
                t2 = t0 + 2
                pltpu.make_async_copy(ps_hbm.at[pl.ds(a_j, SLAB)],
                                      buf_a, sem_a).wait()

                @pl.when(t1 < nslab)
                def _():
                    pltpu.async_copy(
                        ps_hbm.at[pl.ds(a_j + t1 * SLAB, SLAB)], buf_b, sem_b)

                lo0, hi0 = clip(t0)
                carries = _accum_rows(buf_a, lo0, hi0, carries)

                @pl.when(t2 < nslab)
                def _():
                    pltpu.async_copy(
                        ps_hbm.at[pl.ds(a_j + t2 * SLAB, SLAB)], buf_a, sem_a)

                @pl.when(t1 < nslab)
                def _():
                    pltpu.make_async_copy(ps_hbm.at[pl.ds(a_j, SLAB)],
                                          buf_b, sem_b).wait()

                lo1, hi1 = clip(t1)
                return _accum_rows(buf_b, lo1, hi1, carries)

            carries = lax.fori_loop(0, npair, pair_body, carries)
            for cc in range(NCH):
                acc[j, pl.ds(cc * 16, 16)] = carries[cc]

        @pl.when(nslab <= 0)
        def _(carries=carries, j=j):
            for cc in range(NCH):
                acc[j, pl.ds(cc * 16, 16)] = carries[cc]

    pltpu.sync_copy(acc, out_hbm.at[pl.ds(GPW * w, GPW)])


def _sc_segment_sum(pairsum, edges, starts):
    mesh = plsc.VectorSubcoreMesh(core_axis_name="c", subcore_axis_name="s",
                                  num_cores=2, num_subcores=16)
    k = pl.kernel(
        _sc_body,
        out_type=jax.ShapeDtypeStruct((NUM_GRAPHS, HP), jnp.float32),
        mesh=mesh,
        scratch_types=[
            pltpu.VMEM((NUM_GRAPHS + 8,), jnp.float32),
            pltpu.VMEM((SLAB, HP), jnp.float32),
            pltpu.VMEM((SLAB, HP), jnp.float32),
            pltpu.VMEM((16, HP), jnp.float32),
            pltpu.VMEM((GPW, HP), jnp.float32),
            pltpu.SemaphoreType.DMA,
            pltpu.SemaphoreType.DMA,
        ],
    )
    return k(pairsum, edges, starts)


def _final_body(p_ref, w_ref, b_ref, o_ref):
    o_ref[...] = jnp.dot(p_ref[...], w_ref[...],
                         preferred_element_type=jnp.float32) + b_ref[...]


def _final(sums, W_func, b_func):
    return pl.pallas_call(
        _final_body,
        out_shape=jax.ShapeDtypeStruct((NUM_GRAPHS, HP), jnp.float32),
    )(sums, W_func, b_func.reshape(1, HP))


def kernel(node_features, node_to_graph_id, W_up, b_up, W_gate, b_gate, W_func, b_func):
    ids32 = node_to_graph_id.astype(jnp.int32)
    starts = jnp.searchsorted(ids32, jnp.arange(NUM_GRAPHS + 1, dtype=jnp.int32),
                              side="left").astype(jnp.int32)

    seg_s = starts[:-1]
    seg_e = starts[1:]
    nonempty = seg_s < seg_e
    head_valid = nonempty & (seg_s % 2 == 1)
    tail_valid = nonempty & (seg_e % 2 == 1)
    head_row = jnp.where(head_valid, seg_s, 0)
    tail_row = jnp.where(tail_valid, seg_e - 1, 0)
    edge_rows = jnp.stack([head_row, tail_row], axis=1).reshape(NE)
    edge_mask = jnp.stack([head_valid, tail_valid],
                          axis=1).reshape(NE, 1).astype(jnp.float32)
    x_e = node_features[edge_rows]

    starts_f = jnp.concatenate(
        [starts, jnp.full((7,), N_NODES, jnp.int32)]).astype(jnp.float32)

    pairsum = _mlp_pairsum(node_features, W_up, b_up, W_gate, b_gate)
    edges = _edge_mlp(x_e, W_up, b_up, W_gate, b_gate, edge_mask)
    sums = _sc_segment_sum(pairsum, edges, starts_f)
    return _final(sums, W_func, b_func)


# speedup vs baseline: 1.1023x; 1.0996x over previous
import jax
import jax.numpy as jnp
from jax import lax
from jax.experimental import pallas as pl
from jax.experimental.pallas import tpu as pltpu
from jax.experimental.pallas import tpu_sc as plsc

H = 256
HP = 512
NUM_GRAPHS = 256
N_NODES = 100000
N_PAIRS = N_NODES // 2

ROW_BLOCK = 2000
PAIR_BLOCK = ROW_BLOCK // 2
N_BLOCKS = N_NODES // ROW_BLOCK
PS_PAD = (N_BLOCKS + 1) * PAIR_BLOCK
SLAB = 64
NCH = HP // 16
GPW = NUM_GRAPHS // 32
NE = 2 * NUM_GRAPHS


def _gated(x, wu, bu, wg, bg):
    up = jnp.dot(x, wu, preferred_element_type=jnp.float32) + bu
    gl = jnp.dot(x, wg, preferred_element_type=jnp.float32) + bg
    return up * (1.0 / (1.0 + jnp.exp(-gl)))


def _mlp_body(x_ref, wu_ref, bu_ref, wg_ref, bg_ref, o_ref):
    i = pl.program_id(0)

    @pl.when(i < N_BLOCKS)
    def _():
        res = _gated(x_ref[...], wu_ref[...], bu_ref[...],
                     wg_ref[...], bg_ref[...])
        merged = res.reshape(PAIR_BLOCK, 2 * HP)
        o_ref[...] = merged[:, :HP] + merged[:, HP:]

    @pl.when(i >= N_BLOCKS)
    def _():
        o_ref[...] = jnp.zeros_like(o_ref)


def _mlp_pairsum(x, W_up, b_up, W_gate, b_gate):
    return pl.pallas_call(
        _mlp_body,
        grid=(N_BLOCKS + 1,),
        in_specs=[
            pl.BlockSpec((ROW_BLOCK, H), lambda i: (jnp.minimum(i, N_BLOCKS - 1), 0)),
            pl.BlockSpec((H, HP), lambda i: (0, 0)),
            pl.BlockSpec((1, HP), lambda i: (0, 0)),
            pl.BlockSpec((H, HP), lambda i: (0, 0)),
            pl.BlockSpec((1, HP), lambda i: (0, 0)),
        ],
        out_specs=pl.BlockSpec((PAIR_BLOCK, HP), lambda i: (i, 0)),
        out_shape=jax.ShapeDtypeStruct((PS_PAD, HP), jnp.float32),
    )(x, W_up, b_up.reshape(1, HP), W_gate, b_gate.reshape(1, HP))


def _edge_body(x_ref, wu_ref, bu_ref, wg_ref, bg_ref, m_ref, o_ref):
    res = _gated(x_ref[...], wu_ref[...], bu_ref[...], wg_ref[...], bg_ref[...])
    o_ref[...] = res * m_ref[...]


def _edge_mlp(x_e, W_up, b_up, W_gate, b_gate, mask):
    return pl.pallas_call(
        _edge_body,
        out_shape=jax.ShapeDtypeStruct((NE, HP), jnp.float32),
    )(x_e, W_up, b_up.reshape(1, HP), W_gate, b_gate.reshape(1, HP), mask)


def _accum_rows(buf, lo, hi, carries):
    def row_body(r, cs):
        return tuple(v + buf[r, pl.ds(cc * 16, 16)] for cc, v in enumerate(cs))

    return lax.fori_loop(lo, hi, row_body, carries)


def _sc_body(ps_hbm, edges_hbm, starts_hbm, out_hbm,
             sv, buf_a, buf_b, ebuf, acc, sem_a, sem_b):
    c = lax.axis_index("c")
    s = lax.axis_index("s")
    w = s * 2 + c

    pltpu.sync_copy(starts_hbm, sv)
    pltpu.sync_copy(edges_hbm.at[pl.ds(16 * w, 16)], ebuf)
    bounds = sv[pl.ds(GPW * w, 16)]

    for j in range(GPW):
        s_j = bounds[j].astype(jnp.int32)
        e_j = bounds[j + 1].astype(jnp.int32)
        p_s = (s_j + 1) // 2
        p_e = e_j // 2
        a_j = (p_s // 8) * 8
        nslab = (p_e - a_j + SLAB - 1) // SLAB
        npair = (nslab + 1) // 2

        def clip(t, p_s=p_s, p_e=p_e, a_j=a_j):
            base = a_j + t * SLAB
            return (jnp.clip(p_s - base, 0, SLAB),
                    jnp.clip(p_e - base, 0, SLAB))

        carries = tuple(
            ebuf[2 * j, pl.ds(cc * 16, 16)] + ebuf[2 * j + 1, pl.ds(cc * 16, 16)]
            for cc in range(NCH))

        @pl.when(nslab > 0)
        def _(a_j=a_j, nslab=nslab, npair=npair, clip=clip, carries=carries, j=j):
            pltpu.async_copy(ps_hbm.at[pl.ds(a_j, SLAB)], buf_a, sem_a)

            def pair_body(u, carries):
                t0 = 2 * u
                t1 = t0 + 1
                t2 = t0 + 2
                pltpu.make_async_copy(ps_hbm.at[pl.ds(a_j, SLAB)],
                                      buf_a, sem_a).wait()

                @pl.when(t1 < nslab)
                def _():
                    pltpu.async_copy(
                        ps_hbm.at[pl.ds(a_j + t1 * SLAB, SLAB)], buf_b, sem_b)

                lo0, hi0 = clip(t0)
                carries = _accum_rows(buf_a, lo0, hi0, carries)

                @pl.when(t2 < nslab)
                def _():
                    pltpu.async_copy(
                        ps_hbm.at[pl.ds(a_j + t2 * SLAB, SLAB)], buf_a, sem_a)

                @pl.when(t1 < nslab)
                def _():
                    pltpu.make_async_copy(ps_hbm.at[pl.ds(a_j, SLAB)],
                                          buf_b, sem_b).wait()

                lo1, hi1 = clip(t1)
                return _accum_rows(buf_b, lo1, hi1, carries)

            carries = lax.fori_loop(0, npair, pair_body, carries)
            for cc in range(NCH):
                acc[j, pl.ds(cc * 16, 16)] = carries[cc]

        @pl.when(nslab <= 0)
        def _(carries=carries, j=j):
            for cc in range(NCH):
                acc[j, pl.ds(cc * 16, 16)] = carries[cc]

    pltpu.sync_copy(acc, out_hbm.at[pl.ds(GPW * w, GPW)])


def _sc_segment_sum(pairsum, edges, starts):
    mesh = plsc.VectorSubcoreMesh(core_axis_name="c", subcore_axis_name="s",
                                  num_cores=2, num_subcores=16)
    k = pl.kernel(
        _sc_body,
        out_type=jax.ShapeDtypeStruct((NUM_GRAPHS, HP), jnp.float32),
        mesh=mesh,
        scratch_types=[
            pltpu.VMEM((NUM_GRAPHS + 8,), jnp.float32),
            pltpu.VMEM((SLAB, HP), jnp.float32),
            pltpu.VMEM((SLAB, HP), jnp.float32),
            pltpu.VMEM((16, HP), jnp.float32),
            pltpu.VMEM((GPW, HP), jnp.float32),
            pltpu.SemaphoreType.DMA,
            pltpu.SemaphoreType.DMA,
        ],
    )
    return k(pairsum, edges, starts)


def _final_body(p_ref, w_ref, b_ref, o_ref):
    o_ref[...] = jnp.dot(p_ref[...], w_ref[...],
                         preferred_element_type=jnp.float32) + b_ref[...]


def _final(sums, W_func, b_func):
    return pl.pallas_call(
        _final_body,
        out_shape=jax.ShapeDtypeStruct((NUM_GRAPHS, HP), jnp.float32),
    )(sums, W_func, b_func.reshape(1, HP))


def kernel(node_features, node_to_graph_id, W_up, b_up, W_gate, b_gate, W_func, b_func):
    ids32 = node_to_graph_id.astype(jnp.int32)
    starts = jnp.searchsorted(ids32, jnp.arange(NUM_GRAPHS + 1, dtype=jnp.int32),
                              side="left").astype(jnp.int32)

    seg_s = starts[:-1]
    seg_e = starts[1:]
    nonempty = seg_s < seg_e
    head_valid = nonempty & (seg_s % 2 == 1)
    tail_valid = nonempty & (seg_e % 2 == 1)
    head_row = jnp.where(head_valid, seg_s, 0)
    tail_row = jnp.where(tail_valid, seg_e - 1, 0)
    edge_rows = jnp.stack([head_row, tail_row], axis=1).reshape(NE)
    edge_mask = jnp.stack([head_valid, tail_valid],
                          axis=1).reshape(NE, 1).astype(jnp.float32)
    x_e = node_features[edge_rows]

    starts_f = jnp.concatenate(
        [starts, jnp.full((7,), N_NODES, jnp.int32)]).astype(jnp.float32)

    pairsum = _mlp_pairsum(node_features, W_up, b_up, W_gate, b_gate)
    edges = _edge_mlp(x_e, W_up, b_up, W_gate, b_gate, edge_mask)
    sums = _sc_segment_sum(pairsum, edges, starts_f)
    return _final(sums, W_func, b_func)


# speedup vs baseline: 1.1221x; 1.0180x over previous
import jax
import jax.numpy as jnp
from jax import lax
from jax.experimental import pallas as pl
from jax.experimental.pallas import tpu as pltpu
from jax.experimental.pallas import tpu_sc as plsc

H = 256
HP = 512
NUM_GRAPHS = 256
N_NODES = 100000
N_PAIRS = N_NODES // 2

ROW_BLOCK = 4000
PAIR_BLOCK = ROW_BLOCK // 2
N_BLOCKS = N_NODES // ROW_BLOCK
PS_PAD = (N_BLOCKS + 1) * PAIR_BLOCK
SLAB = 64
NCH = HP // 16
GPW = NUM_GRAPHS // 32
NE = 2 * NUM_GRAPHS


def _gated(x, wu, bu, wg, bg):
    up = jnp.dot(x, wu, preferred_element_type=jnp.float32) + bu
    gl = jnp.dot(x, wg, preferred_element_type=jnp.float32) + bg
    return up * (1.0 / (1.0 + jnp.exp(-gl)))


def _mlp_body(x_ref, wu_ref, bu_ref, wg_ref, bg_ref, o_ref):
    i = pl.program_id(0)

    @pl.when(i < N_BLOCKS)
    def _():
        res = _gated(x_ref[...], wu_ref[...], bu_ref[...],
                     wg_ref[...], bg_ref[...])
        merged = res.reshape(PAIR_BLOCK, 2 * HP)
        o_ref[...] = merged[:, :HP] + merged[:, HP:]

    @pl.when(i >= N_BLOCKS)
    def _():
        o_ref[...] = jnp.zeros_like(o_ref)


def _mlp_pairsum(x, W_up, b_up, W_gate, b_gate):
    return pl.pallas_call(
        _mlp_body,
        grid=(N_BLOCKS + 1,),
        in_specs=[
            pl.BlockSpec((ROW_BLOCK, H), lambda i: (jnp.minimum(i, N_BLOCKS - 1), 0)),
            pl.BlockSpec((H, HP), lambda i: (0, 0)),
            pl.BlockSpec((1, HP), lambda i: (0, 0)),
            pl.BlockSpec((H, HP), lambda i: (0, 0)),
            pl.BlockSpec((1, HP), lambda i: (0, 0)),
        ],
        out_specs=pl.BlockSpec((PAIR_BLOCK, HP), lambda i: (i, 0)),
        out_shape=jax.ShapeDtypeStruct((PS_PAD, HP), jnp.float32),
    )(x, W_up, b_up.reshape(1, HP), W_gate, b_gate.reshape(1, HP))


def _edge_body(x_ref, wu_ref, bu_ref, wg_ref, bg_ref, m_ref, o_ref):
    res = _gated(x_ref[...], wu_ref[...], bu_ref[...], wg_ref[...], bg_ref[...])
    o_ref[...] = res * m_ref[...]


def _edge_mlp(x_e, W_up, b_up, W_gate, b_gate, mask):
    return pl.pallas_call(
        _edge_body,
        out_shape=jax.ShapeDtypeStruct((NE, HP), jnp.float32),
    )(x_e, W_up, b_up.reshape(1, HP), W_gate, b_gate.reshape(1, HP), mask)


def _accum_rows(buf, lo, hi, carries):
    def row_body(r, cs):
        return tuple(v + buf[r, pl.ds(cc * 16, 16)] for cc, v in enumerate(cs))

    return lax.fori_loop(lo, hi, row_body, carries)


def _sc_body(ps_hbm, edges_hbm, starts_hbm, out_hbm,
             sv, buf_a, buf_b, ebuf, acc, sem_a, sem_b):
    c = lax.axis_index("c")
    s = lax.axis_index("s")
    w = s * 2 + c

    pltpu.sync_copy(starts_hbm, sv)
    pltpu.sync_copy(edges_hbm.at[pl.ds(16 * w, 16)], ebuf)
    bounds = sv[pl.ds(GPW * w, 16)]

    for j in range(GPW):
        s_j = bounds[j].astype(jnp.int32)
        e_j = bounds[j + 1].astype(jnp.int32)
        p_s = (s_j + 1) // 2
        p_e = e_j // 2
        a_j = (p_s // 8) * 8
        nslab = (p_e - a_j + SLAB - 1) // SLAB
        npair = (nslab + 1) // 2

        def clip(t, p_s=p_s, p_e=p_e, a_j=a_j):
            base = a_j + t * SLAB
            return (jnp.clip(p_s - base, 0, SLAB),
                    jnp.clip(p_e - base, 0, SLAB))

        carries = tuple(
            ebuf[2 * j, pl.ds(cc * 16, 16)] + ebuf[2 * j + 1, pl.ds(cc * 16, 16)]
            for cc in range(NCH))

        @pl.when(nslab > 0)
        def _(a_j=a_j, nslab=nslab, npair=npair, clip=clip, carries=carries, j=j):
            pltpu.async_copy(ps_hbm.at[pl.ds(a_j, SLAB)], buf_a, sem_a)

            def pair_body(u, carries):
                t0 = 2 * u
                t1 = t0 + 1
                t2 = t0 + 2
                pltpu.make_async_copy(ps_hbm.at[pl.ds(a_j, SLAB)],
                                      buf_a, sem_a).wait()

                @pl.when(t1 < nslab)
                def _():
                    pltpu.async_copy(
                        ps_hbm.at[pl.ds(a_j + t1 * SLAB, SLAB)], buf_b, sem_b)

                lo0, hi0 = clip(t0)
                carries = _accum_rows(buf_a, lo0, hi0, carries)

                @pl.when(t2 < nslab)
                def _():
                    pltpu.async_copy(
                        ps_hbm.at[pl.ds(a_j + t2 * SLAB, SLAB)], buf_a, sem_a)

                @pl.when(t1 < nslab)
                def _():
                    pltpu.make_async_copy(ps_hbm.at[pl.ds(a_j, SLAB)],
                                          buf_b, sem_b).wait()

                lo1, hi1 = clip(t1)
                return _accum_rows(buf_b, lo1, hi1, carries)

            carries = lax.fori_loop(0, npair, pair_body, carries)
            for cc in range(NCH):
                acc[j, pl.ds(cc * 16, 16)] = carries[cc]

        @pl.when(nslab <= 0)
        def _(carries=carries, j=j):
            for cc in range(NCH):
                acc[j, pl.ds(cc * 16, 16)] = carries[cc]

    pltpu.sync_copy(acc, out_hbm.at[pl.ds(GPW * w, GPW)])


def _sc_segment_sum(pairsum, edges, starts):
    mesh = plsc.VectorSubcoreMesh(core_axis_name="c", subcore_axis_name="s",
                                  num_cores=2, num_subcores=16)
    k = pl.kernel(
        _sc_body,
        out_type=jax.ShapeDtypeStruct((NUM_GRAPHS, HP), jnp.float32),
        mesh=mesh,
        scratch_types=[
            pltpu.VMEM((NUM_GRAPHS + 8,), jnp.float32),
            pltpu.VMEM((SLAB, HP), jnp.float32),
            pltpu.VMEM((SLAB, HP), jnp.float32),
            pltpu.VMEM((16, HP), jnp.float32),
            pltpu.VMEM((GPW, HP), jnp.float32),
            pltpu.SemaphoreType.DMA,
            pltpu.SemaphoreType.DMA,
        ],
    )
    return k(pairsum, edges, starts)


def _final_body(p_ref, w_ref, b_ref, o_ref):
    o_ref[...] = jnp.dot(p_ref[...], w_ref[...],
                         preferred_element_type=jnp.float32) + b_ref[...]


def _final(sums, W_func, b_func):
    return pl.pallas_call(
        _final_body,
        out_shape=jax.ShapeDtypeStruct((NUM_GRAPHS, HP), jnp.float32),
    )(sums, W_func, b_func.reshape(1, HP))


def kernel(node_features, node_to_graph_id, W_up, b_up, W_gate, b_gate, W_func, b_func):
    ids32 = node_to_graph_id.astype(jnp.int32)
    starts = jnp.searchsorted(ids32, jnp.arange(NUM_GRAPHS + 1, dtype=jnp.int32),
                              side="left").astype(jnp.int32)

    seg_s = starts[:-1]
    seg_e = starts[1:]
    nonempty = seg_s < seg_e
    head_valid = nonempty & (seg_s % 2 == 1)
    tail_valid = nonempty & (seg_e % 2 == 1)
    head_row = jnp.where(head_valid, seg_s, 0)
    tail_row = jnp.where(tail_valid, seg_e - 1, 0)
    edge_rows = jnp.stack([head_row, tail_row], axis=1).reshape(NE)
    edge_mask = jnp.stack([head_valid, tail_valid],
                          axis=1).reshape(NE, 1).astype(jnp.float32)
    x_e = node_features[edge_rows]

    starts_f = jnp.concatenate(
        [starts, jnp.full((7,), N_NODES, jnp.int32)]).astype(jnp.float32)

    pairsum = _mlp_pairsum(node_features, W_up, b_up, W_gate, b_gate)
    edges = _edge_mlp(x_e, W_up, b_up, W_gate, b_gate, edge_mask)
    sums = _sc_segment_sum(pairsum, edges, starts_f)
    return _final(sums, W_func, b_func)
